# speedup vs baseline: 3.4384x; 3.4384x over previous
import functools

import jax
import jax.numpy as jnp
from jax import lax
from jax.experimental import pallas as pl
from jax.experimental.pallas import tpu as pltpu
from jax.experimental.pallas import tpu_sc as plsc

VOCAB = 90
D = 128
BATCH = 16384
HIST = 200
N = BATCH * HIST
NUM_CORES = 2
NUM_SUBCORES = 16
NW = NUM_CORES * NUM_SUBCORES
PER_W = N // NW
CHUNK = 512
NCHUNK = PER_W // CHUNK

assert PER_W * NW == N
assert NCHUNK * CHUNK == PER_W
assert CHUNK % 8 == 0 and PER_W % 8 == 0


def _build_kernel():
  mesh = plsc.VectorSubcoreMesh(core_axis_name="c", subcore_axis_name="s")

  @functools.partial(
      pl.kernel,
      mesh=mesh,
      out_type=jax.ShapeDtypeStruct((N, D), jnp.float32),
      scratch_types=[
          pltpu.VMEM((CHUNK,), jnp.int32),
          pltpu.VMEM((CHUNK, D), jnp.float32),
          pltpu.SemaphoreType.DMA,
      ],
  )
  def emb_kernel(idx_hbm, table_hbm, out_hbm, idx_v, rows_v, sem):
    wid = lax.axis_index("s") * NUM_CORES + lax.axis_index("c")
    base = wid * PER_W

    def body(i, carry):
      off = base + i * CHUNK
      pltpu.sync_copy(idx_hbm.at[pl.ds(off, CHUNK)], idx_v)
      pltpu.async_copy(table_hbm.at[idx_v], rows_v, sem).wait()
      pltpu.sync_copy(rows_v, out_hbm.at[pl.ds(off, CHUNK)])
      return carry

    lax.fori_loop(0, NCHUNK, body, 0)

  return emb_kernel


_EMB_KERNEL = _build_kernel()


@jax.jit
def kernel(note, table):
  flat = note.reshape(-1)
  out = _EMB_KERNEL(flat, table)
  return out.reshape(BATCH, HIST, D)


# speedup vs baseline: 3.4770x; 1.0112x over previous
import functools

import jax
import jax.numpy as jnp
from jax import lax
from jax.experimental import pallas as pl
from jax.experimental.pallas import tpu as pltpu
from jax.experimental.pallas import tpu_sc as plsc

VOCAB = 90
D = 128
BATCH = 16384
HIST = 200
N = BATCH * HIST
NUM_CORES = 2
NUM_SUBCORES = 16
NW = NUM_CORES * NUM_SUBCORES
PER_W = N // NW
CHUNK = 400
NCHUNK = PER_W // CHUNK

assert PER_W * NW == N
assert NCHUNK * CHUNK == PER_W
assert NCHUNK % 2 == 0
assert CHUNK % 8 == 0 and PER_W % 8 == 0


def _build_kernel():
  mesh = plsc.VectorSubcoreMesh(core_axis_name="c", subcore_axis_name="s")

  @functools.partial(
      pl.kernel,
      mesh=mesh,
      out_type=jax.ShapeDtypeStruct((N, D), jnp.float32),
      scratch_types=[
          pltpu.VMEM((CHUNK,), jnp.int32),
          pltpu.VMEM((CHUNK,), jnp.int32),
          pltpu.VMEM((CHUNK, D), jnp.float32),
          pltpu.VMEM((CHUNK, D), jnp.float32),
          pltpu.SemaphoreType.DMA,
          pltpu.SemaphoreType.DMA,
          pltpu.SemaphoreType.DMA,
          pltpu.SemaphoreType.DMA,
          pltpu.SemaphoreType.DMA,
          pltpu.SemaphoreType.DMA,
      ],
  )
  def emb_kernel(idx_hbm, table_hbm, out_hbm, idx0, idx1, rows0, rows1,
                 gsem0, gsem1, osem0, osem1, isem0, isem1):
    wid = lax.axis_index("s") * NUM_CORES + lax.axis_index("c")
    base = wid * PER_W
    idx_v = (idx0, idx1)
    rows_v = (rows0, rows1)
    gsem = (gsem0, gsem1)
    osem = (osem0, osem1)
    isem = (isem0, isem1)

    def idx_src(i):
      return idx_hbm.at[pl.ds(base + i * CHUNK, CHUNK)]

    def out_dst(i):
      return out_hbm.at[pl.ds(base + i * CHUNK, CHUNK)]

    pltpu.sync_copy(idx_src(0), idx_v[0])
    pltpu.async_copy(table_hbm.at[idx_v[0]], rows_v[0], gsem[0])
    pltpu.sync_copy(idx_src(1), idx_v[1])
    pltpu.async_copy(table_hbm.at[idx_v[1]], rows_v[1], gsem[1])
    pltpu.make_async_copy(table_hbm.at[idx_v[0]], rows_v[0], gsem[0]).wait()
    pltpu.async_copy(rows_v[0], out_dst(0), osem[0])
    pltpu.async_copy(idx_src(2), idx_v[0], isem[0])

    def body(g, carry):
      for b in (0, 1):
        i = 2 * g + b
        pltpu.make_async_copy(rows_v[b], out_dst(i), osem[b]).wait()
        pltpu.make_async_copy(idx_src(i), idx_v[b], isem[b]).wait()
        pltpu.async_copy(table_hbm.at[idx_v[b]], rows_v[b], gsem[b])
        pltpu.make_async_copy(
            table_hbm.at[idx_v[1 - b]], rows_v[1 - b], gsem[1 - b]).wait()
        nxt = jnp.minimum(i + 1, NCHUNK - 1)
        pltpu.async_copy(idx_src(nxt), idx_v[1 - b], isem[1 - b])
        pltpu.async_copy(rows_v[1 - b], out_dst(i - 1), osem[1 - b])
      return carry

    lax.fori_loop(1, NCHUNK // 2, body, 0)

    last = NCHUNK - 1
    pltpu.make_async_copy(table_hbm.at[idx_v[1]], rows_v[1], gsem[1]).wait()
    pltpu.async_copy(rows_v[1], out_dst(last), osem[1])
    pltpu.make_async_copy(idx_src(last), idx_v[0], isem[0]).wait()
    pltpu.make_async_copy(rows_v[0], out_dst(last - 1), osem[0]).wait()
    pltpu.make_async_copy(rows_v[1], out_dst(last), osem[1]).wait()

  return emb_kernel


_EMB_KERNEL = _build_kernel()


@jax.jit
def kernel(note, table):
  flat = note.reshape(-1)
  out = _EMB_KERNEL(flat, table)
  return out.reshape(BATCH, HIST, D)


# speedup vs baseline: 19.1397x; 5.5047x over previous
import functools

import jax
import jax.numpy as jnp
from jax import lax
from jax.experimental import pallas as pl
from jax.experimental.pallas import tpu as pltpu
from jax.experimental.pallas import tpu_sc as plsc

VOCAB = 90
D = 128
BATCH = 16384
HIST = 200
N = BATCH * HIST
NUM_CORES = 2
NUM_SUBCORES = 16
NW = NUM_CORES * NUM_SUBCORES
PER_W = N // NW
CHUNK = 400
NCHUNK = PER_W // CHUNK

assert PER_W * NW == N
assert NCHUNK * CHUNK == PER_W
assert NCHUNK % 2 == 0
assert CHUNK % 8 == 0 and PER_W % 8 == 0


def _build_kernel():
  mesh = plsc.VectorSubcoreMesh(core_axis_name="c", subcore_axis_name="s")

  @functools.partial(
      pl.kernel,
      mesh=mesh,
      out_type=jax.ShapeDtypeStruct((N, D), jnp.float32),
      scratch_types=[
          pltpu.VMEM_SHARED((VOCAB, D), jnp.float32),
          pltpu.VMEM((CHUNK,), jnp.int32),
          pltpu.VMEM((CHUNK,), jnp.int32),
          pltpu.VMEM((CHUNK, D), jnp.float32),
          pltpu.VMEM((CHUNK, D), jnp.float32),
          pltpu.SemaphoreType.DMA,
          pltpu.SemaphoreType.DMA,
          pltpu.SemaphoreType.DMA,
          pltpu.SemaphoreType.DMA,
          pltpu.SemaphoreType.DMA,
          pltpu.SemaphoreType.DMA,
      ],
  )
  def emb_kernel(idx_hbm, table_hbm, out_hbm, shared_tab, idx0, idx1,
                 rows0, rows1, gsem0, gsem1, osem0, osem1, isem0, isem1):
    sid = lax.axis_index("s")
    wid = sid * NUM_CORES + lax.axis_index("c")
    base = wid * PER_W

    @pl.when(sid == 0)
    def _stage():
      pltpu.sync_copy(table_hbm, shared_tab)

    plsc.subcore_barrier()
    idx_v = (idx0, idx1)
    rows_v = (rows0, rows1)
    gsem = (gsem0, gsem1)
    osem = (osem0, osem1)
    isem = (isem0, isem1)

    def idx_src(i):
      return idx_hbm.at[pl.ds(base + i * CHUNK, CHUNK)]

    def out_dst(i):
      return out_hbm.at[pl.ds(base + i * CHUNK, CHUNK)]

    pltpu.sync_copy(idx_src(0), idx_v[0])
    pltpu.async_copy(shared_tab.at[idx_v[0]], rows_v[0], gsem[0])
    pltpu.sync_copy(idx_src(1), idx_v[1])
    pltpu.async_copy(shared_tab.at[idx_v[1]], rows_v[1], gsem[1])
    pltpu.make_async_copy(shared_tab.at[idx_v[0]], rows_v[0], gsem[0]).wait()
    pltpu.async_copy(rows_v[0], out_dst(0), osem[0])
    pltpu.async_copy(idx_src(2), idx_v[0], isem[0])

    def body(g, carry):
      for b in (0, 1):
        i = 2 * g + b
        pltpu.make_async_copy(rows_v[b], out_dst(i), osem[b]).wait()
        pltpu.make_async_copy(idx_src(i), idx_v[b], isem[b]).wait()
        pltpu.async_copy(shared_tab.at[idx_v[b]], rows_v[b], gsem[b])
        pltpu.make_async_copy(
            shared_tab.at[idx_v[1 - b]], rows_v[1 - b], gsem[1 - b]).wait()
        nxt = jnp.minimum(i + 1, NCHUNK - 1)
        pltpu.async_copy(idx_src(nxt), idx_v[1 - b], isem[1 - b])
        pltpu.async_copy(rows_v[1 - b], out_dst(i - 1), osem[1 - b])
      return carry

    lax.fori_loop(1, NCHUNK // 2, body, 0)

    last = NCHUNK - 1
    pltpu.make_async_copy(shared_tab.at[idx_v[1]], rows_v[1], gsem[1]).wait()
    pltpu.async_copy(rows_v[1], out_dst(last), osem[1])
    pltpu.make_async_copy(idx_src(last), idx_v[0], isem[0]).wait()
    pltpu.make_async_copy(rows_v[0], out_dst(last - 1), osem[0]).wait()
    pltpu.make_async_copy(rows_v[1], out_dst(last), osem[1]).wait()

  return emb_kernel


_EMB_KERNEL = _build_kernel()


@jax.jit
def kernel(note, table):
  flat = note.reshape(-1)
  out = _EMB_KERNEL(flat, table)
  return out.reshape(BATCH, HIST, D)
